# Initial kernel scaffold; baseline (speedup 1.0000x reference)
import jax
import jax.numpy as jnp
from jax.experimental import pallas as pl


def kernel(hidden, y, table, sampling_probs):
    raise NotImplementedError("write your pallas kernel here")


# speedup vs baseline: 1.2133x; 1.2133x over previous
import functools

import jax
import jax.numpy as jnp
from jax.experimental import pallas as pl
from jax.experimental.pallas import tpu as pltpu

_VOCAB = 1000000
_DIM = 32
_BATCH = 16384
_N_NEG = 4096
_TEMP = 1.0


def _fused_loss(hidden, y3, e_pos, e_neg, tp3, sp3, si3):
    B, D = hidden.shape
    N = e_neg.shape[0]
    BB = 512
    G = B // BB

    def body(h_ref, y_ref, ep_ref, en_ref, tp_ref, sp_ref, si_ref, out_ref,
             acc_ref):
        i = pl.program_id(0)

        @pl.when(i == 0)
        def _init():
            acc_ref[0] = 0.0
            acc_ref[1] = 0.0

        h = h_ref[...]
        en = en_ref[...]
        logits = jax.lax.dot_general(h, en, (((1,), (1,)), ((), ())),
                                     preferred_element_type=jnp.float32)
        yb = y_ref[0, 0, :]
        si = si_ref[0, 0, :]
        sp = sp_ref[0, 0, :]
        tp = tp_ref[0, 0, :]
        coll = yb[:, None] == si[None, :]
        neg = (jnp.where(coll, -1e9, logits) / _TEMP
               - jnp.log(sp + 1e-10)[None, :])
        pos = jnp.sum(h * ep_ref[...], axis=1) / _TEMP - jnp.log(tp + 1e-10)
        m = jnp.maximum(jnp.max(neg, axis=1), pos)
        s = jnp.sum(jnp.exp(neg - m[:, None]), axis=1) + jnp.exp(pos - m)
        per_row = m + jnp.log(s) - pos
        maskf = (yb != 0).astype(jnp.float32)
        acc_ref[0] += jnp.sum(per_row * maskf)
        acc_ref[1] += jnp.sum(maskf)

        @pl.when(i == G - 1)
        def _fin():
            out_ref[...] = jnp.reshape(acc_ref[0] / acc_ref[1], (1, 1))

    out = pl.pallas_call(
        body,
        grid=(G,),
        in_specs=[
            pl.BlockSpec((BB, D), lambda i: (i, 0)),
            pl.BlockSpec((1, 1, BB), lambda i: (i, 0, 0)),
            pl.BlockSpec((BB, D), lambda i: (i, 0)),
            pl.BlockSpec((N, D), lambda i: (0, 0)),
            pl.BlockSpec((1, 1, BB), lambda i: (i, 0, 0)),
            pl.BlockSpec((1, 1, N), lambda i: (0, 0, 0)),
            pl.BlockSpec((1, 1, N), lambda i: (0, 0, 0)),
        ],
        out_specs=pl.BlockSpec((1, 1), lambda i: (0, 0)),
        out_shape=jax.ShapeDtypeStruct((1, 1), jnp.float32),
        scratch_shapes=[pltpu.SMEM((2,), jnp.float32)],
    )(hidden, y3, e_pos, e_neg, tp3, sp3, si3)
    return out[0, 0]


def kernel(hidden, y, table, sampling_probs):
    hidden = hidden.reshape(-1, hidden.shape[-1])
    y = y.reshape(-1)
    B, D = hidden.shape
    N = _N_NEG

    cdf = jnp.cumsum(sampling_probs)
    u = jax.random.uniform(jax.random.key(42), (N,), dtype=jnp.float32)
    sampled = jnp.clip(jnp.searchsorted(cdf, u), 0, _VOCAB - 1).astype(jnp.int32)
    tp = sampling_probs[y]
    sp = sampling_probs[sampled]
    e_pos = jnp.take(table, y, axis=0)
    e_neg = jnp.take(table, sampled, axis=0)

    BB = 512
    G = B // BB
    y3 = y.reshape(G, 1, BB)
    tp3 = tp.reshape(G, 1, BB)
    sp3 = sp.reshape(1, 1, N)
    si3 = sampled.reshape(1, 1, N)
    return _fused_loss(hidden, y3, e_pos, e_neg, tp3, sp3, si3)
